# baseline (speedup 1.0000x reference)
import jax
import jax.numpy as jnp
from jax import lax
from jax.experimental import pallas as pl
from jax.experimental.pallas import tpu as pltpu
from jax.experimental.pallas import tpu_sc as plsc

NUM_CORES = 2
NUM_SUBCORES = 16
NUM_WORKERS = NUM_CORES * NUM_SUBCORES
TOKENS = 32768
D_MODEL = 2048
SEGS = 16
ROWS = 8
SC_TOKENS = 6144
TC_BLK = 2048
TPW = SC_TOKENS // NUM_WORKERS
NCHUNK = TPW // ROWS
LANES = 16
KSLICES = D_MODEL // LANES


def _sc_pool_body(hid_hbm, lens_hbm, part_hbm,
                  buf0, buf1, lens_v, acc, row_v, shared, sem0, sem1):
    cid = lax.axis_index("c")
    sid = lax.axis_index("s")
    wid = cid * NUM_SUBCORES + sid
    base = wid * TPW

    pltpu.sync_copy(lens_hbm, lens_v)
    lv = lens_v[...]
    ends = []
    e = jnp.int32(0)
    for s in range(SEGS):
        e = e + lv[s]
        ends.append(lax.broadcast_in_dim(e, (LANES,), ()))

    z = jnp.zeros((LANES,), jnp.float32)

    @pl.loop(0, SEGS * KSLICES // 16)
    def _(j):
        for u in range(16):
            acc[pl.ds((j * 16 + u) * LANES, LANES)] = z

    iota16 = lax.iota(jnp.int32, LANES)
    bufs = (buf0, buf1)
    sems = (sem0, sem1)

    pltpu.async_copy(hid_hbm.at[pl.ds(base, ROWS)], buf0, sem0)
    pltpu.async_copy(hid_hbm.at[pl.ds(base + ROWS, ROWS)], buf1, sem1)

    @pl.loop(0, NCHUNK, step=2)
    def _(c):
        for b in range(2):
            cur = c + b
            buf, sem = bufs[b], sems[b]
            pltpu.make_async_copy(hid_hbm.at[pl.ds(0, ROWS)], buf, sem).wait()
            tok = base + cur * ROWS + iota16
            seg = jnp.zeros((LANES,), jnp.int32)
            for e in ends:
                seg = seg + (tok >= e).astype(jnp.int32)
            seg0 = seg[0]
            uniform = seg0 == seg[ROWS - 1]

            @pl.when(uniform)
            def _fast():
                segoff = seg0 * D_MODEL

                @pl.loop(0, KSLICES)
                def _(k):
                    o = k * LANES
                    v = [buf[r, pl.ds(o, LANES)] for r in range(ROWS)]
                    while len(v) > 1:
                        v = [v[i] + v[i + 1] for i in range(0, len(v) - 1, 2)] \
                            + ([v[-1]] if len(v) % 2 else [])
                    plsc.addupdate(acc.at[pl.ds(segoff + o, LANES)], v[0])

            @pl.when(jnp.logical_not(uniform))
            def _slow():
                for r in range(ROWS):
                    segoff = seg[r] * D_MODEL

                    @pl.loop(0, KSLICES)
                    def _(k):
                        o = k * LANES
                        plsc.addupdate(acc.at[pl.ds(segoff + o, LANES)],
                                       buf[r, pl.ds(o, LANES)])

            @pl.when(cur + 2 < NCHUNK)
            def _next():
                pltpu.async_copy(
                    hid_hbm.at[pl.ds(base + (cur + 2) * ROWS, ROWS)], buf, sem)

    pltpu.sync_copy(acc, shared.at[sid])
    plsc.subcore_barrier()
    pltpu.sync_copy(
        shared.at[pl.ds(0, ROWS), pl.ds(sid * D_MODEL, D_MODEL)], buf0)
    pltpu.sync_copy(
        shared.at[pl.ds(ROWS, ROWS), pl.ds(sid * D_MODEL, D_MODEL)], buf1)

    @pl.loop(0, KSLICES)
    def _(k):
        o = k * LANES
        v = ([buf0[r, pl.ds(o, LANES)] for r in range(ROWS)]
             + [buf1[r, pl.ds(o, LANES)] for r in range(ROWS)])
        while len(v) > 1:
            v = [v[i] + v[i + 1] for i in range(0, len(v) - 1, 2)] \
                + ([v[-1]] if len(v) % 2 else [])
        row_v[pl.ds(o, LANES)] = v[0]

    pltpu.sync_copy(row_v, part_hbm.at[cid, sid])


_sc_pool = pl.kernel(
    _sc_pool_body,
    out_type=jax.ShapeDtypeStruct((NUM_CORES, SEGS, D_MODEL), jnp.float32),
    mesh=plsc.VectorSubcoreMesh(
        core_axis_name="c", subcore_axis_name="s",
        num_cores=NUM_CORES, num_subcores=NUM_SUBCORES),
    compiler_params=pltpu.CompilerParams(needs_layout_passes=False),
    scratch_types=[
        pltpu.VMEM((ROWS, D_MODEL), jnp.float32),
        pltpu.VMEM((ROWS, D_MODEL), jnp.float32),
        pltpu.VMEM((SEGS,), jnp.int32),
        pltpu.VMEM((SEGS * D_MODEL,), jnp.float32),
        pltpu.VMEM((D_MODEL,), jnp.float32),
        pltpu.VMEM_SHARED((NUM_SUBCORES, SEGS * D_MODEL), jnp.float32),
        pltpu.SemaphoreType.DMA,
        pltpu.SemaphoreType.DMA,
    ],
)


def _tc_partial_body(lens_ref, hid_ref, out_ref):
    i = pl.program_id(0)
    lens_f = lens_ref[...].astype(jnp.float32)
    tri = (lax.broadcasted_iota(jnp.int32, (SEGS, SEGS), 0)
           <= lax.broadcasted_iota(jnp.int32, (SEGS, SEGS), 1)
           ).astype(jnp.float32)
    ends = lax.dot_general(lens_f, tri, (((1,), (0,)), ((), ())),
                           preferred_element_type=jnp.float32)
    base = SC_TOKENS + i * TC_BLK
    rid = (base + lax.broadcasted_iota(jnp.int32, (TC_BLK, SEGS), 0)
           ).astype(jnp.float32)
    ends_b = jnp.broadcast_to(ends, (TC_BLK, SEGS))
    starts_b = ends_b - jnp.broadcast_to(lens_f, (TC_BLK, SEGS))
    oh = ((rid >= starts_b) & (rid < ends_b)).astype(jnp.float32)
    part = lax.dot_general(oh, hid_ref[...], (((0,), (0,)), ((), ())),
                           preferred_element_type=jnp.float32)

    @pl.when(i == 0)
    def _init():
        out_ref[...] = part

    @pl.when(i > 0)
    def _acc():
        out_ref[...] += part


def _finalize_body(part_ref, tc_ref, lens_ref, out_ref):
    out_ref[...] = (part_ref[0] + part_ref[1] + tc_ref[...]) / lens_ref[...]


def kernel(hidden_states, prompt_lens):
    lens_i32 = prompt_lens.astype(jnp.int32)
    part = _sc_pool(hidden_states, lens_i32)
    tc_part = pl.pallas_call(
        _tc_partial_body,
        grid=((TOKENS - SC_TOKENS) // TC_BLK,),
        in_specs=[
            pl.BlockSpec((1, SEGS), lambda i: (0, 0)),
            pl.BlockSpec((TC_BLK, D_MODEL), lambda i: (SC_TOKENS // TC_BLK + i, 0)),
        ],
        out_specs=pl.BlockSpec((SEGS, D_MODEL), lambda i: (0, 0)),
        out_shape=jax.ShapeDtypeStruct((SEGS, D_MODEL), jnp.float32),
    )(lens_i32.reshape(1, SEGS), hidden_states)
    lens_f = lens_i32.astype(jnp.float32).reshape(SEGS, 1)
    return pl.pallas_call(
        _finalize_body,
        out_shape=jax.ShapeDtypeStruct((SEGS, D_MODEL), jnp.float32),
    )(part, tc_part, lens_f)


# speedup vs baseline: 1.0080x; 1.0080x over previous
import jax
import jax.numpy as jnp
from jax import lax
from jax.experimental import pallas as pl
from jax.experimental.pallas import tpu as pltpu
from jax.experimental.pallas import tpu_sc as plsc

NUM_CORES = 2
NUM_SUBCORES = 16
NUM_WORKERS = NUM_CORES * NUM_SUBCORES
TOKENS = 32768
D_MODEL = 2048
SEGS = 16
ROWS = 8
SC_TOKENS = 2048
TC_BLK = 2048
TPW = SC_TOKENS // NUM_WORKERS
NCHUNK = TPW // ROWS
LANES = 16
KSLICES = D_MODEL // LANES


def _sc_pool_body(hid_hbm, lens_hbm, part_hbm,
                  buf0, buf1, lens_v, acc, row_v, shared, sem0, sem1):
    cid = lax.axis_index("c")
    sid = lax.axis_index("s")
    wid = cid * NUM_SUBCORES + sid
    base = wid * TPW

    pltpu.sync_copy(lens_hbm, lens_v)
    lv = lens_v[...]
    ends = []
    e = jnp.int32(0)
    for s in range(SEGS):
        e = e + lv[s]
        ends.append(lax.broadcast_in_dim(e, (LANES,), ()))

    z = jnp.zeros((LANES,), jnp.float32)

    @pl.loop(0, SEGS * KSLICES // 16)
    def _(j):
        for u in range(16):
            acc[pl.ds((j * 16 + u) * LANES, LANES)] = z

    iota16 = lax.iota(jnp.int32, LANES)
    bufs = (buf0, buf1)
    sems = (sem0, sem1)

    pltpu.async_copy(hid_hbm.at[pl.ds(base, ROWS)], buf0, sem0)
    pltpu.async_copy(hid_hbm.at[pl.ds(base + ROWS, ROWS)], buf1, sem1)

    @pl.loop(0, NCHUNK, step=2)
    def _(c):
        for b in range(2):
            cur = c + b
            buf, sem = bufs[b], sems[b]
            pltpu.make_async_copy(hid_hbm.at[pl.ds(0, ROWS)], buf, sem).wait()
            tok = base + cur * ROWS + iota16
            seg = jnp.zeros((LANES,), jnp.int32)
            for e in ends:
                seg = seg + (tok >= e).astype(jnp.int32)
            seg0 = seg[0]
            uniform = seg0 == seg[ROWS - 1]

            @pl.when(uniform)
            def _fast():
                segoff = seg0 * D_MODEL

                @pl.loop(0, KSLICES)
                def _(k):
                    o = k * LANES
                    v = [buf[r, pl.ds(o, LANES)] for r in range(ROWS)]
                    while len(v) > 1:
                        v = [v[i] + v[i + 1] for i in range(0, len(v) - 1, 2)] \
                            + ([v[-1]] if len(v) % 2 else [])
                    plsc.addupdate(acc.at[pl.ds(segoff + o, LANES)], v[0])

            @pl.when(jnp.logical_not(uniform))
            def _slow():
                for r in range(ROWS):
                    segoff = seg[r] * D_MODEL

                    @pl.loop(0, KSLICES)
                    def _(k):
                        o = k * LANES
                        plsc.addupdate(acc.at[pl.ds(segoff + o, LANES)],
                                       buf[r, pl.ds(o, LANES)])

            @pl.when(cur + 2 < NCHUNK)
            def _next():
                pltpu.async_copy(
                    hid_hbm.at[pl.ds(base + (cur + 2) * ROWS, ROWS)], buf, sem)

    pltpu.sync_copy(acc, shared.at[sid])
    plsc.subcore_barrier()
    pltpu.sync_copy(
        shared.at[pl.ds(0, ROWS), pl.ds(sid * D_MODEL, D_MODEL)], buf0)
    pltpu.sync_copy(
        shared.at[pl.ds(ROWS, ROWS), pl.ds(sid * D_MODEL, D_MODEL)], buf1)

    @pl.loop(0, KSLICES)
    def _(k):
        o = k * LANES
        v = ([buf0[r, pl.ds(o, LANES)] for r in range(ROWS)]
             + [buf1[r, pl.ds(o, LANES)] for r in range(ROWS)])
        while len(v) > 1:
            v = [v[i] + v[i + 1] for i in range(0, len(v) - 1, 2)] \
                + ([v[-1]] if len(v) % 2 else [])
        row_v[pl.ds(o, LANES)] = v[0]

    pltpu.sync_copy(row_v, part_hbm.at[cid, sid])


_sc_pool = pl.kernel(
    _sc_pool_body,
    out_type=jax.ShapeDtypeStruct((NUM_CORES, SEGS, D_MODEL), jnp.float32),
    mesh=plsc.VectorSubcoreMesh(
        core_axis_name="c", subcore_axis_name="s",
        num_cores=NUM_CORES, num_subcores=NUM_SUBCORES),
    compiler_params=pltpu.CompilerParams(needs_layout_passes=False),
    scratch_types=[
        pltpu.VMEM((ROWS, D_MODEL), jnp.float32),
        pltpu.VMEM((ROWS, D_MODEL), jnp.float32),
        pltpu.VMEM((SEGS,), jnp.int32),
        pltpu.VMEM((SEGS * D_MODEL,), jnp.float32),
        pltpu.VMEM((D_MODEL,), jnp.float32),
        pltpu.VMEM_SHARED((NUM_SUBCORES, SEGS * D_MODEL), jnp.float32),
        pltpu.SemaphoreType.DMA,
        pltpu.SemaphoreType.DMA,
    ],
)


def _tc_partial_body(lens_ref, hid_ref, out_ref):
    i = pl.program_id(0)
    lens_f = lens_ref[...].astype(jnp.float32)
    tri = (lax.broadcasted_iota(jnp.int32, (SEGS, SEGS), 0)
           <= lax.broadcasted_iota(jnp.int32, (SEGS, SEGS), 1)
           ).astype(jnp.float32)
    ends = lax.dot_general(lens_f, tri, (((1,), (0,)), ((), ())),
                           preferred_element_type=jnp.float32)
    base = SC_TOKENS + i * TC_BLK
    rid = (base + lax.broadcasted_iota(jnp.int32, (TC_BLK, SEGS), 0)
           ).astype(jnp.float32)
    ends_b = jnp.broadcast_to(ends, (TC_BLK, SEGS))
    starts_b = ends_b - jnp.broadcast_to(lens_f, (TC_BLK, SEGS))
    oh = ((rid >= starts_b) & (rid < ends_b)).astype(jnp.float32)
    part = lax.dot_general(oh, hid_ref[...], (((0,), (0,)), ((), ())),
                           preferred_element_type=jnp.float32)

    @pl.when(i == 0)
    def _init():
        out_ref[...] = part

    @pl.when(i > 0)
    def _acc():
        out_ref[...] += part


def _finalize_body(part_ref, tc_ref, lens_ref, out_ref):
    out_ref[...] = (part_ref[0] + part_ref[1] + tc_ref[...]) / lens_ref[...]


def kernel(hidden_states, prompt_lens):
    lens_i32 = prompt_lens.astype(jnp.int32)
    part = _sc_pool(hidden_states, lens_i32)
    tc_part = pl.pallas_call(
        _tc_partial_body,
        grid=((TOKENS - SC_TOKENS) // TC_BLK,),
        in_specs=[
            pl.BlockSpec((1, SEGS), lambda i: (0, 0)),
            pl.BlockSpec((TC_BLK, D_MODEL), lambda i: (SC_TOKENS // TC_BLK + i, 0)),
        ],
        out_specs=pl.BlockSpec((SEGS, D_MODEL), lambda i: (0, 0)),
        out_shape=jax.ShapeDtypeStruct((SEGS, D_MODEL), jnp.float32),
    )(lens_i32.reshape(1, SEGS), hidden_states)
    lens_f = lens_i32.astype(jnp.float32).reshape(SEGS, 1)
    return pl.pallas_call(
        _finalize_body,
        out_shape=jax.ShapeDtypeStruct((SEGS, D_MODEL), jnp.float32),
    )(part, tc_part, lens_f)
